# baseline (speedup 1.0000x reference)
import functools

import jax
import jax.numpy as jnp
from jax import lax
from jax.experimental import pallas as pl
from jax.experimental.pallas import tpu as pltpu
from jax.experimental.pallas import tpu_sc as plsc

N = 50000
NP = 50176
E = 800000
EC = 192
EPT = 50112
EP = 801792
HID = 64
NG = 256
PG = 272
HALF = 25088
SPR = 25216
GARB = 25088
NC, NS = 2, 16
BR = 3136
F32 = jnp.float32


def _sc_mesh():
    return plsc.VectorSubcoreMesh(
        core_axis_name="c", subcore_axis_name="s",
        num_cores=NC, num_subcores=NS)


def _i16():
    return lax.iota(jnp.int32, 16)


def _zero_accum(zb_hbm, accum, s):
    zbase = s * (SPR // NS)

    def zb(k, carry):
        pltpu.sync_copy(zb_hbm, accum.at[pl.ds(zbase + k * 128, 128)])
        return carry

    lax.fori_loop(0, 12, zb, 0)
    pltpu.sync_copy(zb_hbm.at[pl.ds(0, 40)],
                    accum.at[pl.ds(zbase + 12 * 128, 40)])


def _writeback(accum, out_hbm, c, s):
    wpt = HALF // NS
    pltpu.sync_copy(accum.at[pl.ds(s * wpt, wpt)],
                    out_hbm.at[pl.ds(c * HALF + s * wpt, wpt)])


def _embed_body(fi_hbm, emb_hbm, zb_hbm, out_hbm,
                accum, ibuf0, ibuf1, gbuf0, gbuf1, gbuf2, nidx,
                sem, ssem):
    c = lax.axis_index("c")
    s = lax.axis_index("s")
    _zero_accum(zb_hbm, accum, s)
    plsc.subcore_barrier()
    nchunks = 12 + jnp.where(s < 4, 1, 0)
    gb = [gbuf0, gbuf1, gbuf2]
    ib = [ibuf0, ibuf1]

    def load_ibuf(j, dst):
        ci = s + NS * j
        pltpu.sync_copy(fi_hbm.at[:, pl.ds(c * HALF + ci * 128, 128)], dst)

    load_ibuf(0, ibuf0)

    def wait_gather(rbuf):
        pltpu.make_async_copy(emb_hbm.at[pl.ds(0, 128)], rbuf, sem).wait()

    def wait_scatter(rbuf):
        pltpu.make_async_copy(zb_hbm, rbuf, ssem).wait()

    def chunk_pair(t, carry):
        for p in range(2):
            j = 2 * t + p

            @pl.when(j < nchunks)
            def _():
                ci = s + NS * j
                lbase = ci * 128
                for k in range(8):
                    nidx[pl.ds(k * 16, 16)] = lbase + k * 16 + _i16()

                @pl.when(j + 1 < nchunks)
                def _():
                    load_ibuf(j + 1, ib[1 - p])
                cur = ib[p]
                pltpu.async_copy(emb_hbm.at[cur.at[0]], gb[0], sem)
                pltpu.async_copy(emb_hbm.at[cur.at[1]], gb[1], sem)
                for i in range(9):
                    wait_gather(gb[i % 3])
                    if i >= 1:
                        wait_scatter(gb[(i + 2) % 3])
                    if i + 2 < 9:
                        pltpu.async_copy(
                            emb_hbm.at[cur.at[i + 2]], gb[(i + 2) % 3], sem)
                    pltpu.async_copy(gb[i % 3], accum.at[nidx], ssem,
                                     add=True)
                wait_scatter(gb[0])
        return carry

    lax.fori_loop(0, 7, chunk_pair, 0)
    plsc.subcore_barrier()
    _writeback(accum, out_hbm, c, s)


def _embed(fi, embflat, zb):
    fn = pl.kernel(
        _embed_body,
        out_type=jax.ShapeDtypeStruct((NP, HID), F32),
        mesh=_sc_mesh(),
        compiler_params=pltpu.CompilerParams(use_tc_tiling_on_sc=False),
        scratch_types=[
            pltpu.VMEM_SHARED((SPR, HID), F32),
            pltpu.VMEM((9, 128), jnp.int32),
            pltpu.VMEM((9, 128), jnp.int32),
            pltpu.VMEM((128, HID), F32),
            pltpu.VMEM((128, HID), F32),
            pltpu.VMEM((128, HID), F32),
            pltpu.VMEM((128,), jnp.int32),
            pltpu.SemaphoreType.DMA,
            pltpu.SemaphoreType.DMA,
        ],
    )
    return fn(fi, embflat, zb)


def _agg_body(feat_hbm, eidx_hbm, zb_hbm, out_hbm,
              accum, rows0, rows1, ebuf0, ebuf1,
              didx0, didx1, gsem, isem, ssem):
    c = lax.axis_index("c")
    s = lax.axis_index("s")
    _zero_accum(zb_hbm, accum, s)
    nch = EPT // EC
    ebase = s * EPT
    base_off = c * HALF
    rows = [rows0, rows1]
    ebuf = [ebuf0, ebuf1]
    didx = [didx0, didx1]
    plsc.subcore_barrier()

    def start_idx(m, dst):
        pltpu.async_copy(eidx_hbm.at[:, pl.ds(ebase + m * EC, EC)], dst, isem)

    def wait_idx(dst):
        pltpu.make_async_copy(eidx_hbm.at[:, pl.ds(0, EC)], dst, isem).wait()

    def start_gather(eb, rbuf):
        pltpu.async_copy(feat_hbm.at[eb.at[0]], rbuf, gsem)

    def wait_gather(rbuf):
        pltpu.make_async_copy(feat_hbm.at[pl.ds(0, EC)], rbuf, gsem).wait()

    def start_scatter(rbuf, dref):
        pltpu.async_copy(rbuf, accum.at[dref], ssem, add=True)

    def wait_scatter(rbuf):
        pltpu.make_async_copy(feat_hbm.at[pl.ds(0, EC)], rbuf, ssem).wait()

    def compute_didx(eb, dref):
        for k in range(EC // 16):
            d = eb[1, pl.ds(k * 16, 16)]
            loc = d - base_off
            oob = (loc < 0) | (loc >= HALF)
            dref[pl.ds(k * 16, 16)] = jnp.where(oob, GARB, loc)

    pltpu.sync_copy(eidx_hbm.at[:, pl.ds(ebase, EC)], ebuf0)
    start_gather(ebuf0, rows0)
    start_idx(1, ebuf1)

    def phase(m, p):
        @pl.when(m + 1 < nch)
        def _():
            wait_idx(ebuf[1 - p])
        wait_gather(rows[p])

        @pl.when(m >= 1)
        def _():
            wait_scatter(rows[1 - p])

        @pl.when(m + 1 < nch)
        def _():
            start_gather(ebuf[1 - p], rows[1 - p])
        compute_didx(ebuf[p], didx[p])

        @pl.when(m + 2 < nch)
        def _():
            start_idx(m + 2, ebuf[p])
        start_scatter(rows[p], didx[p])

    def pair(t, carry):
        phase(2 * t, 0)
        phase(2 * t + 1, 1)
        return carry

    lax.fori_loop(0, nch // 2, pair, 0)
    phase(nch - 1, 0)
    wait_scatter(rows0)
    plsc.subcore_barrier()
    _writeback(accum, out_hbm, c, s)


def _edge_agg(feat, eidx, zb):
    fn = pl.kernel(
        _agg_body,
        out_type=jax.ShapeDtypeStruct((NP, HID), F32),
        mesh=_sc_mesh(),
        compiler_params=pltpu.CompilerParams(use_tc_tiling_on_sc=False),
        scratch_types=[
            pltpu.VMEM_SHARED((SPR, HID), F32),
            pltpu.VMEM((EC, HID), F32),
            pltpu.VMEM((EC, HID), F32),
            pltpu.VMEM((2, EC), jnp.int32),
            pltpu.VMEM((2, EC), jnp.int32),
            pltpu.VMEM((EC,), jnp.int32),
            pltpu.VMEM((EC,), jnp.int32),
            pltpu.SemaphoreType.DMA,
            pltpu.SemaphoreType.DMA,
            pltpu.SemaphoreType.DMA,
        ],
    )
    return fn(feat, eidx, zb)


def _pool_body(g_hbm, l_hbm, b_hbm, zb_hbm, out_hbm,
               gacc, lacc, grow0, grow1, lrow0, lrow1, bidx0, bidx1,
               lsem, ssem):
    c = lax.axis_index("c")
    s = lax.axis_index("s")
    w = s * NC + c
    pltpu.sync_copy(zb_hbm.at[pl.ds(0, 17)], gacc.at[pl.ds(s * 17, 17)])
    pltpu.sync_copy(zb_hbm.at[pl.ds(0, 17)], lacc.at[pl.ds(s * 17, 17)])
    plsc.subcore_barrier()
    nchunks = 12 + jnp.where(w < 8, 1, 0)
    grow = [grow0, grow1]
    lrow = [lrow0, lrow1]
    bidx = [bidx0, bidx1]

    def start_loads(j, p):
        base = (w + 32 * j) * 128
        pltpu.async_copy(b_hbm.at[pl.ds(base, 128)], bidx[p], lsem)
        pltpu.async_copy(g_hbm.at[pl.ds(base, 128)], grow[p], lsem)
        pltpu.async_copy(l_hbm.at[pl.ds(base, 128)], lrow[p], lsem)

    def wait_loads(p):
        pltpu.make_async_copy(b_hbm.at[pl.ds(0, 128)], bidx[p], lsem).wait()
        pltpu.make_async_copy(g_hbm.at[pl.ds(0, 128)], grow[p], lsem).wait()
        pltpu.make_async_copy(l_hbm.at[pl.ds(0, 128)], lrow[p], lsem).wait()

    def drain_scatters(p):
        pltpu.make_async_copy(zb_hbm, grow[p], ssem).wait()
        pltpu.make_async_copy(zb_hbm, lrow[p], ssem).wait()

    start_loads(0, 0)

    def chunk_pair(t, carry):
        for p in range(2):
            j = 2 * t + p

            @pl.when(j < nchunks)
            def _():
                wait_loads(p)

                @pl.when(j >= 1)
                def _():
                    drain_scatters(1 - p)

                @pl.when(j + 1 < nchunks)
                def _():
                    start_loads(j + 1, 1 - p)
                pltpu.async_copy(grow[p], gacc.at[bidx[p]], ssem, add=True)
                pltpu.async_copy(lrow[p], lacc.at[bidx[p]], ssem, add=True)
        return carry

    lax.fori_loop(0, 7, chunk_pair, 0)
    drain_scatters(0)
    plsc.subcore_barrier()
    pltpu.sync_copy(gacc.at[pl.ds(s * 17, 17)],
                    out_hbm.at[0, c, pl.ds(s * 17, 17)])
    pltpu.sync_copy(lacc.at[pl.ds(s * 17, 17)],
                    out_hbm.at[1, c, pl.ds(s * 17, 17)])


def _pool(feat_g, feat_l, batch_p, zb):
    fn = pl.kernel(
        _pool_body,
        out_type=jax.ShapeDtypeStruct((2, NC, PG, HID), F32),
        mesh=_sc_mesh(),
        compiler_params=pltpu.CompilerParams(use_tc_tiling_on_sc=False),
        scratch_types=[
            pltpu.VMEM_SHARED((PG, HID), F32),
            pltpu.VMEM_SHARED((PG, HID), F32),
            pltpu.VMEM((128, HID), F32),
            pltpu.VMEM((128, HID), F32),
            pltpu.VMEM((128, HID), F32),
            pltpu.VMEM((128, HID), F32),
            pltpu.VMEM((128,), jnp.int32),
            pltpu.VMEM((128,), jnp.int32),
            pltpu.SemaphoreType.DMA,
            pltpu.SemaphoreType.DMA,
        ],
    )
    return fn(feat_g, feat_l, batch_p, zb)


def _mm1_body(h_ref, a_ref, w_ref, b_ref, z_ref, st_ref):
    i = pl.program_id(0)
    t = h_ref[...] + a_ref[...]
    z = jnp.dot(t, w_ref[...], preferred_element_type=F32) + b_ref[0][None, :]
    z_ref[...] = z
    rows = i * BR + lax.broadcasted_iota(jnp.int32, (BR, 1), 0)
    zm = jnp.where(rows < N, z, 0.0)
    st = jnp.concatenate(
        [jnp.sum(zm, axis=0)[None], jnp.sum(zm * zm, axis=0)[None],
         jnp.zeros((6, 128), F32)], axis=0)

    @pl.when(i == 0)
    def _():
        st_ref[...] = jnp.zeros_like(st_ref)

    st_ref[...] += st


def _mm1(h, agg, w1cat, b1row):
    return pl.pallas_call(
        _mm1_body,
        grid=(NP // BR,),
        in_specs=[
            pl.BlockSpec((BR, HID), lambda i: (i, 0)),
            pl.BlockSpec((BR, HID), lambda i: (i, 0)),
            pl.BlockSpec((HID, 128), lambda i: (0, 0)),
            pl.BlockSpec((8, 128), lambda i: (0, 0)),
        ],
        out_specs=[
            pl.BlockSpec((BR, 128), lambda i: (i, 0)),
            pl.BlockSpec((8, 128), lambda i: (0, 0)),
        ],
        out_shape=[
            jax.ShapeDtypeStruct((NP, 128), F32),
            jax.ShapeDtypeStruct((8, 128), F32),
        ],
    )(h, agg, w1cat, b1row)


def _norm_relu_mm2(z, st, sm, w2):
    mean = st[0] / float(N)
    var = st[1] / float(N) - mean * mean
    inv = lax.rsqrt(var + 1e-5)
    a = jnp.maximum(
        (z - mean[None, :]) * inv[None, :] * sm[0][None, :] + sm[1][None, :],
        0.0)
    return jnp.dot(a, w2, preferred_element_type=F32) + sm[2][None, :]


def _cos_sum(xl, xg):
    num = jnp.sum(xl * xg, axis=1, keepdims=True)
    den = (jnp.sqrt(jnp.sum(xl * xl, axis=1, keepdims=True))
           * jnp.sqrt(jnp.sum(xg * xg, axis=1, keepdims=True)))
    return jnp.maximum(num / jnp.maximum(den, 1e-8), 0.0)


def _fin1_body(z_ref, st_ref, sm_ref, w2_ref, cs_ref, xc_ref, xl_ref, d1_ref):
    i = pl.program_id(0)
    o = _norm_relu_mm2(z_ref[...], st_ref[...], sm_ref[...], w2_ref[...])
    og = o[:, :HID]
    ol = o[:, HID:]
    cs = 0.9 * og + 0.1 * ol
    cs_ref[...] = cs
    xcv = jnp.maximum(cs, 0.0)
    xc_ref[...] = xcv
    xl_ref[...] = xcv
    rows = i * BR + lax.broadcasted_iota(jnp.int32, (BR, 1), 0)
    contrib = jnp.sum(jnp.where(rows < N, _cos_sum(ol, og), 0.0))

    @pl.when(i == 0)
    def _():
        d1_ref[0, 0] = 0.0

    d1_ref[0, 0] += contrib / float(N)


def _fin1(z, stats, smalls, w2blk):
    return pl.pallas_call(
        _fin1_body,
        grid=(NP // BR,),
        in_specs=[
            pl.BlockSpec((BR, 128), lambda i: (i, 0)),
            pl.BlockSpec((8, 128), lambda i: (0, 0)),
            pl.BlockSpec((8, 128), lambda i: (0, 0)),
            pl.BlockSpec((128, 128), lambda i: (0, 0)),
        ],
        out_specs=[
            pl.BlockSpec((BR, HID), lambda i: (i, 0)),
            pl.BlockSpec((BR, HID), lambda i: (i, 0)),
            pl.BlockSpec((BR, HID), lambda i: (i, 0)),
            pl.BlockSpec((1, 1), lambda i: (0, 0), memory_space=pltpu.SMEM),
        ],
        out_shape=[
            jax.ShapeDtypeStruct((N, HID), F32),
            jax.ShapeDtypeStruct((NP, HID), F32),
            jax.ShapeDtypeStruct((N, HID), F32),
            jax.ShapeDtypeStruct((1, 1), F32),
        ],
    )(z, stats, smalls, w2blk)


def _fin2_body(z_ref, st_ref, sm_ref, w2_ref, og_ref, ol_ref):
    o = _norm_relu_mm2(z_ref[...], st_ref[...], sm_ref[...], w2_ref[...])
    og_ref[...] = o[:, :HID]
    ol_ref[...] = o[:, HID:]


def _fin2(z, stats, smalls, w2blk):
    return pl.pallas_call(
        _fin2_body,
        grid=(NP // BR,),
        in_specs=[
            pl.BlockSpec((BR, 128), lambda i: (i, 0)),
            pl.BlockSpec((8, 128), lambda i: (0, 0)),
            pl.BlockSpec((8, 128), lambda i: (0, 0)),
            pl.BlockSpec((128, 128), lambda i: (0, 0)),
        ],
        out_specs=[
            pl.BlockSpec((BR, HID), lambda i: (i, 0)),
            pl.BlockSpec((BR, HID), lambda i: (i, 0)),
        ],
        out_shape=[
            jax.ShapeDtypeStruct((NP, HID), F32),
            jax.ShapeDtypeStruct((NP, HID), F32),
        ],
    )(z, stats, smalls, w2blk)


def _head_body(p_ref, w_ref, sm_ref, cw_ref,
               pg_ref, plo_ref, cs2_ref, xg3_ref, xl3_ref, cs3_ref, out_ref,
               d2_ref, d3_ref):
    p = p_ref[...]
    pg = p[0:256] + p[272:528]
    plo = p[544:800] + p[816:1072]
    pg_ref[...] = pg
    plo_ref[...] = plo
    d2_ref[0, 0] = jnp.sum(_cos_sum(plo, pg)) / float(NG)
    cs2 = 0.9 * pg + 0.1 * plo
    cs2_ref[...] = cs2
    hh = jnp.maximum(
        jnp.dot(cs2, w_ref[...], preferred_element_type=F32)
        + sm_ref[0][None, :], 0.0)
    xg3 = hh[:, :HID]
    xl3 = hh[:, HID:]
    xg3_ref[...] = xg3
    xl3_ref[...] = xl3
    d3_ref[0, 0] = jnp.sum(_cos_sum(xl3, xg3)) / float(NG)
    cs3 = 0.9 * xg3 + 0.1 * xl3
    cs3_ref[...] = cs3
    out_ref[...] = (jnp.dot(cs3, cw_ref[...], preferred_element_type=F32)
                    + sm_ref[1][None, :])


def _head(pools2d, headw, headsm, clfw):
    big = pl.BlockSpec((2 * NC * PG, HID), lambda: (0, 0))
    g64 = pl.BlockSpec((NG, HID), lambda: (0, 0))
    return pl.pallas_call(
        _head_body,
        grid=(),
        in_specs=[
            big,
            pl.BlockSpec((HID, 128), lambda: (0, 0)),
            pl.BlockSpec((8, 128), lambda: (0, 0)),
            pl.BlockSpec((HID, 128), lambda: (0, 0)),
        ],
        out_specs=[
            g64, g64, g64, g64, g64, g64,
            pl.BlockSpec((NG, 128), lambda: (0, 0)),
            pl.BlockSpec((1, 1), lambda: (0, 0), memory_space=pltpu.SMEM),
            pl.BlockSpec((1, 1), lambda: (0, 0), memory_space=pltpu.SMEM),
        ],
        out_shape=[
            jax.ShapeDtypeStruct((NG, HID), F32),
            jax.ShapeDtypeStruct((NG, HID), F32),
            jax.ShapeDtypeStruct((NG, HID), F32),
            jax.ShapeDtypeStruct((NG, HID), F32),
            jax.ShapeDtypeStruct((NG, HID), F32),
            jax.ShapeDtypeStruct((NG, HID), F32),
            jax.ShapeDtypeStruct((NG, 128), F32),
            jax.ShapeDtypeStruct((1, 1), F32),
            jax.ShapeDtypeStruct((1, 1), F32),
        ],
    )(pools2d, headw, headsm, clfw)


def kernel(x, edge_index, batch, params):
    p = params
    fi = x.T.astype(jnp.int32) + (jnp.arange(9, dtype=jnp.int32) * 200)[:, None]
    fi = jnp.pad(fi, ((0, 0), (0, NP - N)), constant_values=1800)
    embflat = jnp.pad(p['emb'].reshape(1800, HID), ((0, 8), (0, 0)))
    pad_e = jnp.stack([jnp.zeros((EP - E,), jnp.int32),
                       jnp.full((EP - E,), -1, jnp.int32)])
    eidx = jnp.concatenate([edge_index, pad_e], axis=1)
    batch_p = jnp.pad(batch, (0, NP - N), constant_values=NG)
    zb = jnp.zeros((128, HID), F32)

    def pack_layer(g, l):
        w1 = jnp.concatenate([p[g + '_W1'], p[l + '_W1']], axis=1)
        b1 = jnp.zeros((8, 128), F32).at[0].set(
            jnp.concatenate([p[g + '_b1'], p[l + '_b1']]))
        sm = jnp.zeros((8, 128), F32)
        sm = sm.at[0].set(jnp.concatenate([p[g + '_gamma'], p[l + '_gamma']]))
        sm = sm.at[1].set(jnp.concatenate([p[g + '_beta'], p[l + '_beta']]))
        sm = sm.at[2].set(jnp.concatenate([p[g + '_b2'], p[l + '_b2']]))
        w2 = jnp.zeros((128, 128), F32)
        w2 = w2.at[:HID, :HID].set(p[g + '_W2']).at[HID:, HID:].set(p[l + '_W2'])
        return w1, b1, sm, w2

    w1a, b1a, sma, w2a = pack_layer('g1', 'l1')
    w1b, b1b, smb, w2b = pack_layer('g2', 'l2')
    headw = jnp.concatenate([p['lin_W'], p['loc_W']], axis=1)
    headsm = jnp.zeros((8, 128), F32)
    headsm = headsm.at[0].set(jnp.concatenate([p['lin_b'], p['loc_b']]))
    headsm = headsm.at[1, :16].set(p['clf_b'])
    clfw = jnp.zeros((HID, 128), F32).at[:, :16].set(p['clf_W'])

    h = _embed(fi, embflat, zb)
    agg1 = _edge_agg(h, eidx, zb)
    z1, st1 = _mm1(h, agg1, w1a, b1a)
    cs1s, xc, xl1, d1 = _fin1(z1, st1, sma, w2a)
    agg2 = _edge_agg(xc, eidx, zb)
    z2, st2 = _mm1(xc, agg2, w1b, b1b)
    o2g, o2l = _fin2(z2, st2, smb, w2b)
    pools = _pool(o2g, o2l, batch_p, zb)
    pgp, plp, cs2, xg3, xl3, cs3, outp, d2, d3 = _head(
        pools.reshape(2 * NC * PG, HID), headw, headsm, clfw)

    alpha = jnp.array([[0.9, 0.1], [0.9, 0.1]], F32)
    out = outp[:, :16]
    d1s = d1.reshape(())
    d2s = d2.reshape(())
    d3s = d3.reshape(())
    return (out, alpha, alpha, alpha, d1s, d2s, d3s,
            xl1, plp, xl3,
            xl1, pgp, xg3,
            cs1s, cs2, cs3,
            cs1s, cs2, cs3)


# speedup vs baseline: 1.0399x; 1.0399x over previous
import functools

import jax
import jax.numpy as jnp
from jax import lax
from jax.experimental import pallas as pl
from jax.experimental.pallas import tpu as pltpu
from jax.experimental.pallas import tpu_sc as plsc

N = 50000
NP = 50176
E = 800000
EC = 128
EPT = 50048
EP = 800768
HID = 64
NG = 256
PG = 272
HALF = 25088
SPR = 25216
GARB = 25088
NC, NS = 2, 16
BR = 3136
F32 = jnp.float32


def _sc_mesh():
    return plsc.VectorSubcoreMesh(
        core_axis_name="c", subcore_axis_name="s",
        num_cores=NC, num_subcores=NS)


def _i16():
    return lax.iota(jnp.int32, 16)


def _zero_accum(zb_hbm, accum, s):
    zbase = s * (SPR // NS)

    def zb(k, carry):
        pltpu.sync_copy(zb_hbm, accum.at[pl.ds(zbase + k * 128, 128)])
        return carry

    lax.fori_loop(0, 12, zb, 0)
    pltpu.sync_copy(zb_hbm.at[pl.ds(0, 40)],
                    accum.at[pl.ds(zbase + 12 * 128, 40)])


def _writeback(accum, out_hbm, c, s):
    wpt = HALF // NS
    pltpu.sync_copy(accum.at[pl.ds(s * wpt, wpt)],
                    out_hbm.at[pl.ds(c * HALF + s * wpt, wpt)])


def _embed_body(fi_hbm, emb_hbm, zb_hbm, out_hbm,
                accum, ibuf0, ibuf1, gbuf0, gbuf1, gbuf2, nidx,
                sem, ssem):
    c = lax.axis_index("c")
    s = lax.axis_index("s")
    _zero_accum(zb_hbm, accum, s)
    plsc.subcore_barrier()
    nchunks = 12 + jnp.where(s < 4, 1, 0)
    gb = [gbuf0, gbuf1, gbuf2]
    ib = [ibuf0, ibuf1]

    def load_ibuf(j, dst):
        ci = s + NS * j
        pltpu.sync_copy(fi_hbm.at[:, pl.ds(c * HALF + ci * 128, 128)], dst)

    load_ibuf(0, ibuf0)

    def wait_gather(rbuf):
        pltpu.make_async_copy(emb_hbm.at[pl.ds(0, 128)], rbuf, sem).wait()

    def wait_scatter(rbuf):
        pltpu.make_async_copy(zb_hbm, rbuf, ssem).wait()

    def chunk_pair(t, carry):
        for p in range(2):
            j = 2 * t + p

            @pl.when(j < nchunks)
            def _():
                ci = s + NS * j
                lbase = ci * 128
                for k in range(8):
                    nidx[pl.ds(k * 16, 16)] = lbase + k * 16 + _i16()

                @pl.when(j + 1 < nchunks)
                def _():
                    load_ibuf(j + 1, ib[1 - p])
                cur = ib[p]
                pltpu.async_copy(emb_hbm.at[cur.at[0]], gb[0], sem)
                pltpu.async_copy(emb_hbm.at[cur.at[1]], gb[1], sem)
                for i in range(9):
                    wait_gather(gb[i % 3])
                    if i >= 1:
                        wait_scatter(gb[(i + 2) % 3])
                    if i + 2 < 9:
                        pltpu.async_copy(
                            emb_hbm.at[cur.at[i + 2]], gb[(i + 2) % 3], sem)
                    pltpu.async_copy(gb[i % 3], accum.at[nidx], ssem,
                                     add=True)
                wait_scatter(gb[0])
        return carry

    lax.fori_loop(0, 7, chunk_pair, 0)
    plsc.subcore_barrier()
    _writeback(accum, out_hbm, c, s)


def _embed(fi, embflat, zb):
    fn = pl.kernel(
        _embed_body,
        out_type=jax.ShapeDtypeStruct((NP, HID), F32),
        mesh=_sc_mesh(),
        compiler_params=pltpu.CompilerParams(use_tc_tiling_on_sc=False),
        scratch_types=[
            pltpu.VMEM_SHARED((SPR, HID), F32),
            pltpu.VMEM((9, 128), jnp.int32),
            pltpu.VMEM((9, 128), jnp.int32),
            pltpu.VMEM((128, HID), F32),
            pltpu.VMEM((128, HID), F32),
            pltpu.VMEM((128, HID), F32),
            pltpu.VMEM((128,), jnp.int32),
            pltpu.SemaphoreType.DMA,
            pltpu.SemaphoreType.DMA,
        ],
    )
    return fn(fi, embflat, zb)


def _agg_body(feat_hbm, eidx_hbm, zb_hbm, out_hbm,
              accum, rows0, rows1, rows2, ebuf0, ebuf1,
              didx0, didx1, didx2, gsem, isem, ssem):
    c = lax.axis_index("c")
    s = lax.axis_index("s")
    _zero_accum(zb_hbm, accum, s)
    nch = EPT // EC
    ebase = s * EPT
    base_off = c * HALF
    rows = [rows0, rows1, rows2]
    ebuf = [ebuf0, ebuf1]
    didx = [didx0, didx1, didx2]
    plsc.subcore_barrier()

    def start_idx(m, dst):
        pltpu.async_copy(eidx_hbm.at[:, pl.ds(ebase + m * EC, EC)], dst, isem)

    def wait_idx(dst):
        pltpu.make_async_copy(eidx_hbm.at[:, pl.ds(0, EC)], dst, isem).wait()

    def start_gather(eb, rbuf):
        pltpu.async_copy(feat_hbm.at[eb.at[0]], rbuf, gsem)

    def wait_gather(rbuf):
        pltpu.make_async_copy(feat_hbm.at[pl.ds(0, EC)], rbuf, gsem).wait()

    def start_scatter(rbuf, dref):
        pltpu.async_copy(rbuf, accum.at[dref], ssem, add=True)

    def wait_scatter(rbuf):
        pltpu.make_async_copy(feat_hbm.at[pl.ds(0, EC)], rbuf, ssem).wait()

    def compute_didx(eb, dref):
        for k in range(EC // 16):
            d = eb[1, pl.ds(k * 16, 16)]
            loc = d - base_off
            oob = (loc < 0) | (loc >= HALF)
            dref[pl.ds(k * 16, 16)] = jnp.where(oob, GARB, loc)

    pltpu.sync_copy(eidx_hbm.at[:, pl.ds(ebase, EC)], ebuf0)
    start_gather(ebuf0, rows0)
    start_idx(1, ebuf1)

    def phase(m, p3, p2):
        @pl.when(m + 1 < nch)
        def _():
            wait_idx(ebuf[1 - p2])
        wait_gather(rows[p3])

        @pl.when(m >= 2)
        def _():
            wait_scatter(rows[(p3 + 1) % 3])

        @pl.when(m + 1 < nch)
        def _():
            start_gather(ebuf[1 - p2], rows[(p3 + 1) % 3])
        compute_didx(ebuf[p2], didx[p3])

        @pl.when(m + 2 < nch)
        def _():
            start_idx(m + 2, ebuf[p2])
        start_scatter(rows[p3], didx[p3])

    def six(t, carry):
        for q in range(6):
            phase(6 * t + q, q % 3, q % 2)
        return carry

    lax.fori_loop(0, nch // 6, six, 0)
    for m in range(nch - nch % 6, nch):
        phase(m, m % 3, m % 2)
    wait_scatter(rows0)
    wait_scatter(rows1)
    plsc.subcore_barrier()
    _writeback(accum, out_hbm, c, s)


def _edge_agg(feat, eidx, zb):
    fn = pl.kernel(
        _agg_body,
        out_type=jax.ShapeDtypeStruct((NP, HID), F32),
        mesh=_sc_mesh(),
        compiler_params=pltpu.CompilerParams(use_tc_tiling_on_sc=False),
        scratch_types=[
            pltpu.VMEM_SHARED((SPR, HID), F32),
            pltpu.VMEM((EC, HID), F32),
            pltpu.VMEM((EC, HID), F32),
            pltpu.VMEM((EC, HID), F32),
            pltpu.VMEM((2, EC), jnp.int32),
            pltpu.VMEM((2, EC), jnp.int32),
            pltpu.VMEM((EC,), jnp.int32),
            pltpu.VMEM((EC,), jnp.int32),
            pltpu.VMEM((EC,), jnp.int32),
            pltpu.SemaphoreType.DMA,
            pltpu.SemaphoreType.DMA,
            pltpu.SemaphoreType.DMA,
        ],
    )
    return fn(feat, eidx, zb)


def _pool_body(g_hbm, l_hbm, b_hbm, zb_hbm, out_hbm,
               gacc, lacc, grow0, grow1, lrow0, lrow1, bidx0, bidx1,
               lsem, ssem):
    c = lax.axis_index("c")
    s = lax.axis_index("s")
    w = s * NC + c
    pltpu.sync_copy(zb_hbm.at[pl.ds(0, 17)], gacc.at[pl.ds(s * 17, 17)])
    pltpu.sync_copy(zb_hbm.at[pl.ds(0, 17)], lacc.at[pl.ds(s * 17, 17)])
    plsc.subcore_barrier()
    nchunks = 12 + jnp.where(w < 8, 1, 0)
    grow = [grow0, grow1]
    lrow = [lrow0, lrow1]
    bidx = [bidx0, bidx1]

    def start_loads(j, p):
        base = (w + 32 * j) * 128
        pltpu.async_copy(b_hbm.at[pl.ds(base, 128)], bidx[p], lsem)
        pltpu.async_copy(g_hbm.at[pl.ds(base, 128)], grow[p], lsem)
        pltpu.async_copy(l_hbm.at[pl.ds(base, 128)], lrow[p], lsem)

    def wait_loads(p):
        pltpu.make_async_copy(b_hbm.at[pl.ds(0, 128)], bidx[p], lsem).wait()
        pltpu.make_async_copy(g_hbm.at[pl.ds(0, 128)], grow[p], lsem).wait()
        pltpu.make_async_copy(l_hbm.at[pl.ds(0, 128)], lrow[p], lsem).wait()

    def drain_scatters(p):
        pltpu.make_async_copy(zb_hbm, grow[p], ssem).wait()
        pltpu.make_async_copy(zb_hbm, lrow[p], ssem).wait()

    start_loads(0, 0)

    def chunk_pair(t, carry):
        for p in range(2):
            j = 2 * t + p

            @pl.when(j < nchunks)
            def _():
                wait_loads(p)

                @pl.when(j >= 1)
                def _():
                    drain_scatters(1 - p)

                @pl.when(j + 1 < nchunks)
                def _():
                    start_loads(j + 1, 1 - p)
                pltpu.async_copy(grow[p], gacc.at[bidx[p]], ssem, add=True)
                pltpu.async_copy(lrow[p], lacc.at[bidx[p]], ssem, add=True)
        return carry

    lax.fori_loop(0, 7, chunk_pair, 0)
    drain_scatters(0)
    plsc.subcore_barrier()
    pltpu.sync_copy(gacc.at[pl.ds(s * 17, 17)],
                    out_hbm.at[0, c, pl.ds(s * 17, 17)])
    pltpu.sync_copy(lacc.at[pl.ds(s * 17, 17)],
                    out_hbm.at[1, c, pl.ds(s * 17, 17)])


def _pool(feat_g, feat_l, batch_p, zb):
    fn = pl.kernel(
        _pool_body,
        out_type=jax.ShapeDtypeStruct((2, NC, PG, HID), F32),
        mesh=_sc_mesh(),
        compiler_params=pltpu.CompilerParams(use_tc_tiling_on_sc=False),
        scratch_types=[
            pltpu.VMEM_SHARED((PG, HID), F32),
            pltpu.VMEM_SHARED((PG, HID), F32),
            pltpu.VMEM((128, HID), F32),
            pltpu.VMEM((128, HID), F32),
            pltpu.VMEM((128, HID), F32),
            pltpu.VMEM((128, HID), F32),
            pltpu.VMEM((128,), jnp.int32),
            pltpu.VMEM((128,), jnp.int32),
            pltpu.SemaphoreType.DMA,
            pltpu.SemaphoreType.DMA,
        ],
    )
    return fn(feat_g, feat_l, batch_p, zb)


def _mm1_body(h_ref, a_ref, w_ref, b_ref, z_ref, st_ref):
    i = pl.program_id(0)
    t = h_ref[...] + a_ref[...]
    z = jnp.dot(t, w_ref[...], preferred_element_type=F32) + b_ref[0][None, :]
    z_ref[...] = z
    rows = i * BR + lax.broadcasted_iota(jnp.int32, (BR, 1), 0)
    zm = jnp.where(rows < N, z, 0.0)
    st = jnp.concatenate(
        [jnp.sum(zm, axis=0)[None], jnp.sum(zm * zm, axis=0)[None],
         jnp.zeros((6, 128), F32)], axis=0)

    @pl.when(i == 0)
    def _():
        st_ref[...] = jnp.zeros_like(st_ref)

    st_ref[...] += st


def _mm1(h, agg, w1cat, b1row):
    return pl.pallas_call(
        _mm1_body,
        grid=(NP // BR,),
        in_specs=[
            pl.BlockSpec((BR, HID), lambda i: (i, 0)),
            pl.BlockSpec((BR, HID), lambda i: (i, 0)),
            pl.BlockSpec((HID, 128), lambda i: (0, 0)),
            pl.BlockSpec((8, 128), lambda i: (0, 0)),
        ],
        out_specs=[
            pl.BlockSpec((BR, 128), lambda i: (i, 0)),
            pl.BlockSpec((8, 128), lambda i: (0, 0)),
        ],
        out_shape=[
            jax.ShapeDtypeStruct((NP, 128), F32),
            jax.ShapeDtypeStruct((8, 128), F32),
        ],
    )(h, agg, w1cat, b1row)


def _norm_relu_mm2(z, st, sm, w2):
    mean = st[0] / float(N)
    var = st[1] / float(N) - mean * mean
    inv = lax.rsqrt(var + 1e-5)
    a = jnp.maximum(
        (z - mean[None, :]) * inv[None, :] * sm[0][None, :] + sm[1][None, :],
        0.0)
    return jnp.dot(a, w2, preferred_element_type=F32) + sm[2][None, :]


def _cos_sum(xl, xg):
    num = jnp.sum(xl * xg, axis=1, keepdims=True)
    den = (jnp.sqrt(jnp.sum(xl * xl, axis=1, keepdims=True))
           * jnp.sqrt(jnp.sum(xg * xg, axis=1, keepdims=True)))
    return jnp.maximum(num / jnp.maximum(den, 1e-8), 0.0)


def _fin1_body(z_ref, st_ref, sm_ref, w2_ref, cs_ref, xc_ref, xl_ref, d1_ref):
    i = pl.program_id(0)
    o = _norm_relu_mm2(z_ref[...], st_ref[...], sm_ref[...], w2_ref[...])
    og = o[:, :HID]
    ol = o[:, HID:]
    cs = 0.9 * og + 0.1 * ol
    cs_ref[...] = cs
    xcv = jnp.maximum(cs, 0.0)
    xc_ref[...] = xcv
    xl_ref[...] = xcv
    rows = i * BR + lax.broadcasted_iota(jnp.int32, (BR, 1), 0)
    contrib = jnp.sum(jnp.where(rows < N, _cos_sum(ol, og), 0.0))

    @pl.when(i == 0)
    def _():
        d1_ref[0, 0] = 0.0

    d1_ref[0, 0] += contrib / float(N)


def _fin1(z, stats, smalls, w2blk):
    return pl.pallas_call(
        _fin1_body,
        grid=(NP // BR,),
        in_specs=[
            pl.BlockSpec((BR, 128), lambda i: (i, 0)),
            pl.BlockSpec((8, 128), lambda i: (0, 0)),
            pl.BlockSpec((8, 128), lambda i: (0, 0)),
            pl.BlockSpec((128, 128), lambda i: (0, 0)),
        ],
        out_specs=[
            pl.BlockSpec((BR, HID), lambda i: (i, 0)),
            pl.BlockSpec((BR, HID), lambda i: (i, 0)),
            pl.BlockSpec((BR, HID), lambda i: (i, 0)),
            pl.BlockSpec((1, 1), lambda i: (0, 0), memory_space=pltpu.SMEM),
        ],
        out_shape=[
            jax.ShapeDtypeStruct((N, HID), F32),
            jax.ShapeDtypeStruct((NP, HID), F32),
            jax.ShapeDtypeStruct((N, HID), F32),
            jax.ShapeDtypeStruct((1, 1), F32),
        ],
    )(z, stats, smalls, w2blk)


def _fin2_body(z_ref, st_ref, sm_ref, w2_ref, og_ref, ol_ref):
    o = _norm_relu_mm2(z_ref[...], st_ref[...], sm_ref[...], w2_ref[...])
    og_ref[...] = o[:, :HID]
    ol_ref[...] = o[:, HID:]


def _fin2(z, stats, smalls, w2blk):
    return pl.pallas_call(
        _fin2_body,
        grid=(NP // BR,),
        in_specs=[
            pl.BlockSpec((BR, 128), lambda i: (i, 0)),
            pl.BlockSpec((8, 128), lambda i: (0, 0)),
            pl.BlockSpec((8, 128), lambda i: (0, 0)),
            pl.BlockSpec((128, 128), lambda i: (0, 0)),
        ],
        out_specs=[
            pl.BlockSpec((BR, HID), lambda i: (i, 0)),
            pl.BlockSpec((BR, HID), lambda i: (i, 0)),
        ],
        out_shape=[
            jax.ShapeDtypeStruct((NP, HID), F32),
            jax.ShapeDtypeStruct((NP, HID), F32),
        ],
    )(z, stats, smalls, w2blk)


def _head_body(p_ref, w_ref, sm_ref, cw_ref,
               pg_ref, plo_ref, cs2_ref, xg3_ref, xl3_ref, cs3_ref, out_ref,
               d2_ref, d3_ref):
    p = p_ref[...]
    pg = p[0:256] + p[272:528]
    plo = p[544:800] + p[816:1072]
    pg_ref[...] = pg
    plo_ref[...] = plo
    d2_ref[0, 0] = jnp.sum(_cos_sum(plo, pg)) / float(NG)
    cs2 = 0.9 * pg + 0.1 * plo
    cs2_ref[...] = cs2
    hh = jnp.maximum(
        jnp.dot(cs2, w_ref[...], preferred_element_type=F32)
        + sm_ref[0][None, :], 0.0)
    xg3 = hh[:, :HID]
    xl3 = hh[:, HID:]
    xg3_ref[...] = xg3
    xl3_ref[...] = xl3
    d3_ref[0, 0] = jnp.sum(_cos_sum(xl3, xg3)) / float(NG)
    cs3 = 0.9 * xg3 + 0.1 * xl3
    cs3_ref[...] = cs3
    out_ref[...] = (jnp.dot(cs3, cw_ref[...], preferred_element_type=F32)
                    + sm_ref[1][None, :])


def _head(pools2d, headw, headsm, clfw):
    big = pl.BlockSpec((2 * NC * PG, HID), lambda: (0, 0))
    g64 = pl.BlockSpec((NG, HID), lambda: (0, 0))
    return pl.pallas_call(
        _head_body,
        grid=(),
        in_specs=[
            big,
            pl.BlockSpec((HID, 128), lambda: (0, 0)),
            pl.BlockSpec((8, 128), lambda: (0, 0)),
            pl.BlockSpec((HID, 128), lambda: (0, 0)),
        ],
        out_specs=[
            g64, g64, g64, g64, g64, g64,
            pl.BlockSpec((NG, 128), lambda: (0, 0)),
            pl.BlockSpec((1, 1), lambda: (0, 0), memory_space=pltpu.SMEM),
            pl.BlockSpec((1, 1), lambda: (0, 0), memory_space=pltpu.SMEM),
        ],
        out_shape=[
            jax.ShapeDtypeStruct((NG, HID), F32),
            jax.ShapeDtypeStruct((NG, HID), F32),
            jax.ShapeDtypeStruct((NG, HID), F32),
            jax.ShapeDtypeStruct((NG, HID), F32),
            jax.ShapeDtypeStruct((NG, HID), F32),
            jax.ShapeDtypeStruct((NG, HID), F32),
            jax.ShapeDtypeStruct((NG, 128), F32),
            jax.ShapeDtypeStruct((1, 1), F32),
            jax.ShapeDtypeStruct((1, 1), F32),
        ],
    )(pools2d, headw, headsm, clfw)


def kernel(x, edge_index, batch, params):
    p = params
    fi = x.T.astype(jnp.int32) + (jnp.arange(9, dtype=jnp.int32) * 200)[:, None]
    fi = jnp.pad(fi, ((0, 0), (0, NP - N)), constant_values=1800)
    embflat = jnp.pad(p['emb'].reshape(1800, HID), ((0, 8), (0, 0)))
    pad_e = jnp.stack([jnp.zeros((EP - E,), jnp.int32),
                       jnp.full((EP - E,), -1, jnp.int32)])
    eidx = jnp.concatenate([edge_index, pad_e], axis=1)
    batch_p = jnp.pad(batch, (0, NP - N), constant_values=NG)
    zb = jnp.zeros((128, HID), F32)

    def pack_layer(g, l):
        w1 = jnp.concatenate([p[g + '_W1'], p[l + '_W1']], axis=1)
        b1 = jnp.zeros((8, 128), F32).at[0].set(
            jnp.concatenate([p[g + '_b1'], p[l + '_b1']]))
        sm = jnp.zeros((8, 128), F32)
        sm = sm.at[0].set(jnp.concatenate([p[g + '_gamma'], p[l + '_gamma']]))
        sm = sm.at[1].set(jnp.concatenate([p[g + '_beta'], p[l + '_beta']]))
        sm = sm.at[2].set(jnp.concatenate([p[g + '_b2'], p[l + '_b2']]))
        w2 = jnp.zeros((128, 128), F32)
        w2 = w2.at[:HID, :HID].set(p[g + '_W2']).at[HID:, HID:].set(p[l + '_W2'])
        return w1, b1, sm, w2

    w1a, b1a, sma, w2a = pack_layer('g1', 'l1')
    w1b, b1b, smb, w2b = pack_layer('g2', 'l2')
    headw = jnp.concatenate([p['lin_W'], p['loc_W']], axis=1)
    headsm = jnp.zeros((8, 128), F32)
    headsm = headsm.at[0].set(jnp.concatenate([p['lin_b'], p['loc_b']]))
    headsm = headsm.at[1, :16].set(p['clf_b'])
    clfw = jnp.zeros((HID, 128), F32).at[:, :16].set(p['clf_W'])

    h = _embed(fi, embflat, zb)
    agg1 = _edge_agg(h, eidx, zb)
    z1, st1 = _mm1(h, agg1, w1a, b1a)
    cs1s, xc, xl1, d1 = _fin1(z1, st1, sma, w2a)
    agg2 = _edge_agg(xc, eidx, zb)
    z2, st2 = _mm1(xc, agg2, w1b, b1b)
    o2g, o2l = _fin2(z2, st2, smb, w2b)
    pools = _pool(o2g, o2l, batch_p, zb)
    pgp, plp, cs2, xg3, xl3, cs3, outp, d2, d3 = _head(
        pools.reshape(2 * NC * PG, HID), headw, headsm, clfw)

    alpha = jnp.array([[0.9, 0.1], [0.9, 0.1]], F32)
    out = outp[:, :16]
    d1s = d1.reshape(())
    d2s = d2.reshape(())
    d3s = d3.reshape(())
    return (out, alpha, alpha, alpha, d1s, d2s, d3s,
            xl1, plp, xl3,
            xl1, pgp, xg3,
            cs1s, cs2, cs3,
            cs1s, cs2, cs3)


# speedup vs baseline: 1.0435x; 1.0034x over previous
import jax
import jax.numpy as jnp
from jax import lax
from jax.experimental import pallas as pl
from jax.experimental.pallas import tpu as pltpu
from jax.experimental.pallas import tpu_sc as plsc

N = 50000
NP = 50176
E = 800000
EC = 128
EPT = 50048
EP = 800768
HID = 64
NG = 256
PG = 272
HALF = 25088
SPR = 25216
GARB = 25088
NC, NS = 2, 16
BR = 3136
F32 = jnp.float32


def _sc_mesh():
    return plsc.VectorSubcoreMesh(
        core_axis_name="c", subcore_axis_name="s",
        num_cores=NC, num_subcores=NS)


def _i16():
    return lax.iota(jnp.int32, 16)


def _zero_accum(zb_hbm, accum, s):
    zbase = s * (SPR // NS)

    def zb(k, carry):
        pltpu.sync_copy(zb_hbm, accum.at[pl.ds(zbase + k * 128, 128)])
        return carry

    lax.fori_loop(0, 12, zb, 0)
    pltpu.sync_copy(zb_hbm.at[pl.ds(0, 40)],
                    accum.at[pl.ds(zbase + 12 * 128, 40)])


def _writeback(accum, out_hbm, c, s):
    wpt = HALF // NS
    pltpu.sync_copy(accum.at[pl.ds(s * wpt, wpt)],
                    out_hbm.at[pl.ds(c * HALF + s * wpt, wpt)])


def _embed_body(fi_hbm, emb_hbm, zb_hbm, out_hbm,
                accum, ibuf0, ibuf1, gbuf0, gbuf1, gbuf2, nidx,
                sem, ssem):
    c = lax.axis_index("c")
    s = lax.axis_index("s")
    _zero_accum(zb_hbm, accum, s)
    plsc.subcore_barrier()
    nchunks = 12 + jnp.where(s < 4, 1, 0)
    gb = [gbuf0, gbuf1, gbuf2]
    ib = [ibuf0, ibuf1]

    def load_ibuf(j, dst):
        ci = s + NS * j
        pltpu.sync_copy(fi_hbm.at[:, pl.ds(c * HALF + ci * 128, 128)], dst)

    load_ibuf(0, ibuf0)

    def wait_gather(rbuf):
        pltpu.make_async_copy(emb_hbm.at[pl.ds(0, 128)], rbuf, sem).wait()

    def wait_scatter(rbuf):
        pltpu.make_async_copy(zb_hbm, rbuf, ssem).wait()

    def chunk_pair(t, carry):
        for p in range(2):
            j = 2 * t + p

            @pl.when(j < nchunks)
            def _():
                ci = s + NS * j
                lbase = ci * 128
                for k in range(8):
                    nidx[pl.ds(k * 16, 16)] = lbase + k * 16 + _i16()

                @pl.when(j + 1 < nchunks)
                def _():
                    load_ibuf(j + 1, ib[1 - p])
                cur = ib[p]
                pltpu.async_copy(emb_hbm.at[cur.at[0]], gb[0], sem)
                pltpu.async_copy(emb_hbm.at[cur.at[1]], gb[1], sem)
                for i in range(9):
                    wait_gather(gb[i % 3])
                    if i >= 1:
                        wait_scatter(gb[(i + 2) % 3])
                    if i + 2 < 9:
                        pltpu.async_copy(
                            emb_hbm.at[cur.at[i + 2]], gb[(i + 2) % 3], sem)
                    pltpu.async_copy(gb[i % 3], accum.at[nidx], ssem,
                                     add=True)
                wait_scatter(gb[0])
        return carry

    lax.fori_loop(0, 7, chunk_pair, 0)
    plsc.subcore_barrier()
    _writeback(accum, out_hbm, c, s)


def _embed(fi, embflat, zb):
    fn = pl.kernel(
        _embed_body,
        out_type=jax.ShapeDtypeStruct((NP, HID), F32),
        mesh=_sc_mesh(),
        compiler_params=pltpu.CompilerParams(use_tc_tiling_on_sc=False),
        scratch_types=[
            pltpu.VMEM_SHARED((SPR, HID), F32),
            pltpu.VMEM((9, 128), jnp.int32),
            pltpu.VMEM((9, 128), jnp.int32),
            pltpu.VMEM((128, HID), F32),
            pltpu.VMEM((128, HID), F32),
            pltpu.VMEM((128, HID), F32),
            pltpu.VMEM((128,), jnp.int32),
            pltpu.SemaphoreType.DMA,
            pltpu.SemaphoreType.DMA,
        ],
    )
    return fn(fi, embflat, zb)


def _agg_body(feat_hbm, eidx_hbm, zb_hbm, out_hbm,
              accum, rows0, rows1, rows2, ebuf0, ebuf1,
              didx0, didx1, didx2, gsem, isem, ssem):
    c = lax.axis_index("c")
    s = lax.axis_index("s")
    _zero_accum(zb_hbm, accum, s)
    nch = EPT // EC
    ebase = s * EPT
    base_off = c * HALF
    rows = [rows0, rows1, rows2]
    ebuf = [ebuf0, ebuf1]
    didx = [didx0, didx1, didx2]
    plsc.subcore_barrier()

    def start_idx(m, dst):
        pltpu.async_copy(eidx_hbm.at[:, pl.ds(ebase + m * EC, EC)], dst, isem)

    def wait_idx(dst):
        pltpu.make_async_copy(eidx_hbm.at[:, pl.ds(0, EC)], dst, isem).wait()

    def start_gather(eb, rbuf):
        pltpu.async_copy(feat_hbm.at[eb.at[0]], rbuf, gsem)

    def wait_gather(rbuf):
        pltpu.make_async_copy(feat_hbm.at[pl.ds(0, EC)], rbuf, gsem).wait()

    def start_scatter(rbuf, dref):
        pltpu.async_copy(rbuf, accum.at[dref], ssem, add=True)

    def wait_scatter(rbuf):
        pltpu.make_async_copy(feat_hbm.at[pl.ds(0, EC)], rbuf, ssem).wait()

    def compute_didx(eb, dref):
        for k in range(EC // 16):
            d = eb[1, pl.ds(k * 16, 16)]
            loc = d - base_off
            oob = (loc < 0) | (loc >= HALF)
            dref[pl.ds(k * 16, 16)] = jnp.where(oob, GARB, loc)

    pltpu.sync_copy(eidx_hbm.at[:, pl.ds(ebase, EC)], ebuf0)
    start_gather(ebuf0, rows0)
    start_idx(1, ebuf1)

    def phase(m, p3, p2):
        @pl.when(m + 1 < nch)
        def _():
            wait_idx(ebuf[1 - p2])
        wait_gather(rows[p3])

        @pl.when(m >= 2)
        def _():
            wait_scatter(rows[(p3 + 1) % 3])

        @pl.when(m + 1 < nch)
        def _():
            start_gather(ebuf[1 - p2], rows[(p3 + 1) % 3])
        compute_didx(ebuf[p2], didx[p3])

        @pl.when(m + 2 < nch)
        def _():
            start_idx(m + 2, ebuf[p2])
        start_scatter(rows[p3], didx[p3])

    def six(t, carry):
        for q in range(6):
            phase(6 * t + q, q % 3, q % 2)
        return carry

    lax.fori_loop(0, nch // 6, six, 0)
    for m in range(nch - nch % 6, nch):
        phase(m, m % 3, m % 2)
    wait_scatter(rows0)
    wait_scatter(rows1)
    plsc.subcore_barrier()
    _writeback(accum, out_hbm, c, s)


def _edge_agg(feat, eidx, zb):
    fn = pl.kernel(
        _agg_body,
        out_type=jax.ShapeDtypeStruct((NP, HID), F32),
        mesh=_sc_mesh(),
        compiler_params=pltpu.CompilerParams(use_tc_tiling_on_sc=False),
        scratch_types=[
            pltpu.VMEM_SHARED((SPR, HID), F32),
            pltpu.VMEM((EC, HID), F32),
            pltpu.VMEM((EC, HID), F32),
            pltpu.VMEM((EC, HID), F32),
            pltpu.VMEM((2, EC), jnp.int32),
            pltpu.VMEM((2, EC), jnp.int32),
            pltpu.VMEM((EC,), jnp.int32),
            pltpu.VMEM((EC,), jnp.int32),
            pltpu.VMEM((EC,), jnp.int32),
            pltpu.SemaphoreType.DMA,
            pltpu.SemaphoreType.DMA,
            pltpu.SemaphoreType.DMA,
        ],
    )
    return fn(feat, eidx, zb)


def _pool_body(g_hbm, l_hbm, b_hbm, zb_hbm, out_hbm,
               gacc, lacc, grow0, grow1, lrow0, lrow1, bidx0, bidx1,
               lsem, ssem):
    c = lax.axis_index("c")
    s = lax.axis_index("s")
    w = s * NC + c
    pltpu.sync_copy(zb_hbm.at[pl.ds(0, 17)], gacc.at[pl.ds(s * 17, 17)])
    pltpu.sync_copy(zb_hbm.at[pl.ds(0, 17)], lacc.at[pl.ds(s * 17, 17)])
    plsc.subcore_barrier()
    nchunks = 12 + jnp.where(w < 8, 1, 0)
    grow = [grow0, grow1]
    lrow = [lrow0, lrow1]
    bidx = [bidx0, bidx1]

    def start_loads(j, p):
        base = (w + 32 * j) * 128
        pltpu.async_copy(b_hbm.at[pl.ds(base, 128)], bidx[p], lsem)
        pltpu.async_copy(g_hbm.at[pl.ds(base, 128)], grow[p], lsem)
        pltpu.async_copy(l_hbm.at[pl.ds(base, 128)], lrow[p], lsem)

    def wait_loads(p):
        pltpu.make_async_copy(b_hbm.at[pl.ds(0, 128)], bidx[p], lsem).wait()
        pltpu.make_async_copy(g_hbm.at[pl.ds(0, 128)], grow[p], lsem).wait()
        pltpu.make_async_copy(l_hbm.at[pl.ds(0, 128)], lrow[p], lsem).wait()

    def drain_scatters(p):
        pltpu.make_async_copy(zb_hbm, grow[p], ssem).wait()
        pltpu.make_async_copy(zb_hbm, lrow[p], ssem).wait()

    start_loads(0, 0)

    def chunk_pair(t, carry):
        for p in range(2):
            j = 2 * t + p

            @pl.when(j < nchunks)
            def _():
                wait_loads(p)

                @pl.when(j >= 1)
                def _():
                    drain_scatters(1 - p)

                @pl.when(j + 1 < nchunks)
                def _():
                    start_loads(j + 1, 1 - p)
                pltpu.async_copy(grow[p], gacc.at[bidx[p]], ssem, add=True)
                pltpu.async_copy(lrow[p], lacc.at[bidx[p]], ssem, add=True)
        return carry

    lax.fori_loop(0, 7, chunk_pair, 0)
    drain_scatters(0)
    plsc.subcore_barrier()
    pltpu.sync_copy(gacc.at[pl.ds(s * 17, 17)],
                    out_hbm.at[0, c, pl.ds(s * 17, 17)])
    pltpu.sync_copy(lacc.at[pl.ds(s * 17, 17)],
                    out_hbm.at[1, c, pl.ds(s * 17, 17)])


def _pool(feat_g, feat_l, batch_p, zb):
    fn = pl.kernel(
        _pool_body,
        out_type=jax.ShapeDtypeStruct((2, NC, PG, HID), F32),
        mesh=_sc_mesh(),
        compiler_params=pltpu.CompilerParams(use_tc_tiling_on_sc=False),
        scratch_types=[
            pltpu.VMEM_SHARED((PG, HID), F32),
            pltpu.VMEM_SHARED((PG, HID), F32),
            pltpu.VMEM((128, HID), F32),
            pltpu.VMEM((128, HID), F32),
            pltpu.VMEM((128, HID), F32),
            pltpu.VMEM((128, HID), F32),
            pltpu.VMEM((128,), jnp.int32),
            pltpu.VMEM((128,), jnp.int32),
            pltpu.SemaphoreType.DMA,
            pltpu.SemaphoreType.DMA,
        ],
    )
    return fn(feat_g, feat_l, batch_p, zb)


def _mm1_body(h_ref, a_ref, w_ref, b_ref, z_ref, st_ref):
    i = pl.program_id(0)
    t = h_ref[...] + a_ref[...]
    z = jnp.dot(t, w_ref[...], preferred_element_type=F32) + b_ref[0][None, :]
    z_ref[...] = z
    rows = i * BR + lax.broadcasted_iota(jnp.int32, (BR, 1), 0)
    zm = jnp.where(rows < N, z, 0.0)
    st = jnp.concatenate(
        [jnp.sum(zm, axis=0)[None], jnp.sum(zm * zm, axis=0)[None],
         jnp.zeros((6, 128), F32)], axis=0)

    @pl.when(i == 0)
    def _():
        st_ref[...] = jnp.zeros_like(st_ref)

    st_ref[...] += st


def _mm1(h, agg, w1cat, b1row):
    return pl.pallas_call(
        _mm1_body,
        grid=(NP // BR,),
        in_specs=[
            pl.BlockSpec((BR, HID), lambda i: (i, 0)),
            pl.BlockSpec((BR, HID), lambda i: (i, 0)),
            pl.BlockSpec((HID, 128), lambda i: (0, 0)),
            pl.BlockSpec((8, 128), lambda i: (0, 0)),
        ],
        out_specs=[
            pl.BlockSpec((BR, 128), lambda i: (i, 0)),
            pl.BlockSpec((8, 128), lambda i: (0, 0)),
        ],
        out_shape=[
            jax.ShapeDtypeStruct((NP, 128), F32),
            jax.ShapeDtypeStruct((8, 128), F32),
        ],
    )(h, agg, w1cat, b1row)


def _norm_relu_mm2(z, st, sm, w2):
    mean = st[0] / float(N)
    var = st[1] / float(N) - mean * mean
    inv = lax.rsqrt(var + 1e-5)
    a = jnp.maximum(
        (z - mean[None, :]) * inv[None, :] * sm[0][None, :] + sm[1][None, :],
        0.0)
    return jnp.dot(a, w2, preferred_element_type=F32) + sm[2][None, :]


def _cos_sum(xl, xg):
    num = jnp.sum(xl * xg, axis=1, keepdims=True)
    den = (jnp.sqrt(jnp.sum(xl * xl, axis=1, keepdims=True))
           * jnp.sqrt(jnp.sum(xg * xg, axis=1, keepdims=True)))
    return jnp.maximum(num / jnp.maximum(den, 1e-8), 0.0)


def _fin1_body(z_ref, st_ref, sm_ref, w2_ref, cs_ref, xc_ref, xl_ref, d1_ref):
    i = pl.program_id(0)
    o = _norm_relu_mm2(z_ref[...], st_ref[...], sm_ref[...], w2_ref[...])
    og = o[:, :HID]
    ol = o[:, HID:]
    cs = 0.9 * og + 0.1 * ol
    cs_ref[...] = cs
    xcv = jnp.maximum(cs, 0.0)
    xc_ref[...] = xcv
    xl_ref[...] = xcv
    rows = i * BR + lax.broadcasted_iota(jnp.int32, (BR, 1), 0)
    contrib = jnp.sum(jnp.where(rows < N, _cos_sum(ol, og), 0.0))

    @pl.when(i == 0)
    def _():
        d1_ref[0, 0] = 0.0

    d1_ref[0, 0] += contrib / float(N)


def _fin1(z, stats, smalls, w2blk):
    return pl.pallas_call(
        _fin1_body,
        grid=(NP // BR,),
        in_specs=[
            pl.BlockSpec((BR, 128), lambda i: (i, 0)),
            pl.BlockSpec((8, 128), lambda i: (0, 0)),
            pl.BlockSpec((8, 128), lambda i: (0, 0)),
            pl.BlockSpec((128, 128), lambda i: (0, 0)),
        ],
        out_specs=[
            pl.BlockSpec((BR, HID), lambda i: (i, 0)),
            pl.BlockSpec((BR, HID), lambda i: (i, 0)),
            pl.BlockSpec((BR, HID), lambda i: (i, 0)),
            pl.BlockSpec((1, 1), lambda i: (0, 0), memory_space=pltpu.SMEM),
        ],
        out_shape=[
            jax.ShapeDtypeStruct((N, HID), F32),
            jax.ShapeDtypeStruct((NP, HID), F32),
            jax.ShapeDtypeStruct((N, HID), F32),
            jax.ShapeDtypeStruct((1, 1), F32),
        ],
    )(z, stats, smalls, w2blk)


def _fin2_body(z_ref, st_ref, sm_ref, w2_ref, og_ref, ol_ref):
    o = _norm_relu_mm2(z_ref[...], st_ref[...], sm_ref[...], w2_ref[...])
    og_ref[...] = o[:, :HID]
    ol_ref[...] = o[:, HID:]


def _fin2(z, stats, smalls, w2blk):
    return pl.pallas_call(
        _fin2_body,
        grid=(NP // BR,),
        in_specs=[
            pl.BlockSpec((BR, 128), lambda i: (i, 0)),
            pl.BlockSpec((8, 128), lambda i: (0, 0)),
            pl.BlockSpec((8, 128), lambda i: (0, 0)),
            pl.BlockSpec((128, 128), lambda i: (0, 0)),
        ],
        out_specs=[
            pl.BlockSpec((BR, HID), lambda i: (i, 0)),
            pl.BlockSpec((BR, HID), lambda i: (i, 0)),
        ],
        out_shape=[
            jax.ShapeDtypeStruct((NP, HID), F32),
            jax.ShapeDtypeStruct((NP, HID), F32),
        ],
    )(z, stats, smalls, w2blk)


def _head_body(p_ref, w_ref, sm_ref, cw_ref,
               pg_ref, plo_ref, cs2_ref, xg3_ref, xl3_ref, cs3_ref, out_ref,
               d2_ref, d3_ref):
    p = p_ref[...]
    pg = p[0:256] + p[272:528]
    plo = p[544:800] + p[816:1072]
    pg_ref[...] = pg
    plo_ref[...] = plo
    d2_ref[0, 0] = jnp.sum(_cos_sum(plo, pg)) / float(NG)
    cs2 = 0.9 * pg + 0.1 * plo
    cs2_ref[...] = cs2
    hh = jnp.maximum(
        jnp.dot(cs2, w_ref[...], preferred_element_type=F32)
        + sm_ref[0][None, :], 0.0)
    xg3 = hh[:, :HID]
    xl3 = hh[:, HID:]
    xg3_ref[...] = xg3
    xl3_ref[...] = xl3
    d3_ref[0, 0] = jnp.sum(_cos_sum(xl3, xg3)) / float(NG)
    cs3 = 0.9 * xg3 + 0.1 * xl3
    cs3_ref[...] = cs3
    out_ref[...] = (jnp.dot(cs3, cw_ref[...], preferred_element_type=F32)
                    + sm_ref[1][None, :])


def _head(pools2d, headw, headsm, clfw):
    big = pl.BlockSpec((2 * NC * PG, HID), lambda: (0, 0))
    g64 = pl.BlockSpec((NG, HID), lambda: (0, 0))
    return pl.pallas_call(
        _head_body,
        grid=(),
        in_specs=[
            big,
            pl.BlockSpec((HID, 128), lambda: (0, 0)),
            pl.BlockSpec((8, 128), lambda: (0, 0)),
            pl.BlockSpec((HID, 128), lambda: (0, 0)),
        ],
        out_specs=[
            g64, g64, g64, g64, g64, g64,
            pl.BlockSpec((NG, 128), lambda: (0, 0)),
            pl.BlockSpec((1, 1), lambda: (0, 0), memory_space=pltpu.SMEM),
            pl.BlockSpec((1, 1), lambda: (0, 0), memory_space=pltpu.SMEM),
        ],
        out_shape=[
            jax.ShapeDtypeStruct((NG, HID), F32),
            jax.ShapeDtypeStruct((NG, HID), F32),
            jax.ShapeDtypeStruct((NG, HID), F32),
            jax.ShapeDtypeStruct((NG, HID), F32),
            jax.ShapeDtypeStruct((NG, HID), F32),
            jax.ShapeDtypeStruct((NG, HID), F32),
            jax.ShapeDtypeStruct((NG, 128), F32),
            jax.ShapeDtypeStruct((1, 1), F32),
            jax.ShapeDtypeStruct((1, 1), F32),
        ],
    )(pools2d, headw, headsm, clfw)


def kernel(x, edge_index, batch, params):
    p = params
    fi = x.T.astype(jnp.int32) + (jnp.arange(9, dtype=jnp.int32) * 200)[:, None]
    fi = jnp.pad(fi, ((0, 0), (0, NP - N)), constant_values=1800)
    embflat = jnp.pad(p['emb'].reshape(1800, HID), ((0, 8), (0, 0)))
    pad_e = jnp.stack([jnp.zeros((EP - E,), jnp.int32),
                       jnp.full((EP - E,), -1, jnp.int32)])
    eidx = jnp.concatenate([edge_index, pad_e], axis=1)
    batch_p = jnp.pad(batch, (0, NP - N), constant_values=NG)
    zb = jnp.zeros((128, HID), F32)

    def pack_layer(g, l):
        w1 = jnp.concatenate([p[g + '_W1'], p[l + '_W1']], axis=1)
        b1 = jnp.zeros((8, 128), F32).at[0].set(
            jnp.concatenate([p[g + '_b1'], p[l + '_b1']]))
        sm = jnp.zeros((8, 128), F32)
        sm = sm.at[0].set(jnp.concatenate([p[g + '_gamma'], p[l + '_gamma']]))
        sm = sm.at[1].set(jnp.concatenate([p[g + '_beta'], p[l + '_beta']]))
        sm = sm.at[2].set(jnp.concatenate([p[g + '_b2'], p[l + '_b2']]))
        w2 = jnp.zeros((128, 128), F32)
        w2 = w2.at[:HID, :HID].set(p[g + '_W2']).at[HID:, HID:].set(p[l + '_W2'])
        return w1, b1, sm, w2

    w1a, b1a, sma, w2a = pack_layer('g1', 'l1')
    w1b, b1b, smb, w2b = pack_layer('g2', 'l2')
    headw = jnp.concatenate([p['lin_W'], p['loc_W']], axis=1)
    headsm = jnp.zeros((8, 128), F32)
    headsm = headsm.at[0].set(jnp.concatenate([p['lin_b'], p['loc_b']]))
    headsm = headsm.at[1, :16].set(p['clf_b'])
    clfw = jnp.zeros((HID, 128), F32).at[:, :16].set(p['clf_W'])

    h = _embed(fi, embflat, zb)
    agg1 = _edge_agg(h, eidx, zb)
    z1, st1 = _mm1(h, agg1, w1a, b1a)
    cs1s, xc, xl1, d1 = _fin1(z1, st1, sma, w2a)
    agg2 = _edge_agg(xc, eidx, zb)
    z2, st2 = _mm1(xc, agg2, w1b, b1b)
    o2g, o2l = _fin2(z2, st2, smb, w2b)
    pools = _pool(o2g, o2l, batch_p, zb)
    pgp, plp, cs2, xg3, xl3, cs3, outp, d2, d3 = _head(
        pools.reshape(2 * NC * PG, HID), headw, headsm, clfw)

    alpha = jnp.array([[0.9, 0.1], [0.9, 0.1]], F32)
    out = outp[:, :16]
    d1s = d1.reshape(())
    d2s = d2.reshape(())
    d3s = d3.reshape(())
    return (out, alpha, alpha, alpha, d1s, d2s, d3s,
            xl1, plp, xl3,
            xl1, pgp, xg3,
            cs1s, cs2, cs3,
            cs1s, cs2, cs3)
